# baseline (speedup 1.0000x reference)
import functools

import jax
import jax.numpy as jnp
from jax import lax
from jax.experimental import pallas as pl
from jax.experimental.pallas import tpu as pltpu
from jax.experimental.pallas import tpu_sc as plsc

VOCAB = 1000000
MAX_POS = 8192
DIM = 64
BATCH = 4
SEQ = 2048

_NC = 2
_NS = 16
_NW = _NC * _NS
_N = BATCH * SEQ
_BPW = _N // _NW
_LANES = 16


def _embed_kernel(idx_hbm, table_hbm, pos_hbm, out_hbm, idx_v, rows_v, pos_v,
                  gsem):
    wid = lax.axis_index("s") * _NC + lax.axis_index("c")
    base = wid * _BPW
    pltpu.sync_copy(idx_hbm.at[pl.ds(base, _BPW)], idx_v)
    gather = pltpu.async_copy(table_hbm.at[idx_v], rows_v, gsem)
    s0 = lax.rem(base, SEQ)
    pltpu.sync_copy(pos_hbm.at[pl.ds(s0, _BPW)], pos_v)
    gather.wait()

    def row_body(i, _):
        for j in range(DIM // _LANES):
            sl = (i, pl.ds(j * _LANES, _LANES))
            rows_v[sl] = rows_v[sl] + pos_v[sl]
        return 0

    lax.fori_loop(0, _BPW, row_body, 0)
    pltpu.sync_copy(rows_v, out_hbm.at[pl.ds(base, _BPW)])


def kernel(input_text, token_table, pos_table):
    idx = input_text.reshape(_N).astype(jnp.int32)
    mesh = plsc.VectorSubcoreMesh(core_axis_name="c", subcore_axis_name="s")
    run = functools.partial(
        pl.kernel,
        mesh=mesh,
        compiler_params=pltpu.CompilerParams(use_tc_tiling_on_sc=False),
        out_type=jax.ShapeDtypeStruct((_N, DIM), jnp.float32),
        scratch_types=[
            pltpu.VMEM((_BPW,), jnp.int32),
            pltpu.VMEM((_BPW, DIM), jnp.float32),
            pltpu.VMEM((_BPW, DIM), jnp.float32),
            pltpu.SemaphoreType.DMA,
        ],
    )(_embed_kernel)
    out = run(idx, token_table, pos_table)
    return out.reshape(BATCH, SEQ, DIM)
